# Initial kernel scaffold; baseline (speedup 1.0000x reference)
import jax
import jax.numpy as jnp
from jax.experimental import pallas as pl


def kernel(adj_matrices, ids, W1, b1, W2, b2):
    raise NotImplementedError("write your pallas kernel here")


# speedup vs baseline: 5.2218x; 5.2218x over previous
import functools

import jax
import jax.numpy as jnp
from jax.experimental import pallas as pl
from jax.experimental.pallas import tpu as pltpu


def _body(adj_ref, ids_ref, W1_ref, b1_ref, W2_ref, b2_ref, out_ref,
          carry_ref, w_ref, *, T, NC):
    j = pl.program_id(0)

    @pl.when(j == 0)
    def _():
        carry_ref[...] = jnp.zeros_like(carry_ref)

    H1 = jnp.dot(ids_ref[...], W1_ref[...], preferred_element_type=jnp.float32)
    H2 = jnp.dot(H1, W2_ref[...], preferred_element_type=jnp.float32)
    c = jnp.dot(b1_ref[...], W2_ref[...], preferred_element_type=jnp.float32)

    carry = carry_ref[...]
    for t in range(T - 1, -1, -1):
        carry = 0.5 * (carry + adj_ref[:, t])
        w_ref[:, t] = carry
    carry_ref[...] = carry

    w = w_ref[...]
    Bb, _, Qq, _ = w.shape
    Dd = ids_ref.shape[-1]

    row = jax.lax.broadcasted_iota(jnp.int32, (Qq, Qq), 0)
    col = jax.lax.broadcasted_iota(jnp.int32, (Qq, Qq), 1)
    eye = (row == col).astype(jnp.float32)
    a = w * (1.0 - eye) + eye

    deg = jnp.maximum(jnp.sum(a, axis=-1), 1.0)
    dis = jax.lax.rsqrt(deg)
    a_n = a * dis[..., None] * dis[..., None, :]
    rs = jnp.sum(a_n, axis=-1)

    a_flat = a_n.reshape(Bb * T * Qq, Qq)
    y = jnp.dot(a_flat, H2, preferred_element_type=jnp.float32)
    z = jax.lax.dot_general(
        a_n.reshape(Bb * T, Qq, Qq), y.reshape(Bb * T, Qq, Dd),
        dimension_numbers=(((2,), (1,)), ((0,), (0,))),
        preferred_element_type=jnp.float32)

    out = (z.reshape(Bb, T, Qq, Dd)
           + rs[..., None] * c.reshape(1, 1, 1, Dd)
           + b2_ref[...].reshape(1, 1, 1, Dd))
    out_ref[...] = out


def kernel(adj_matrices, ids, W1, b1, W2, b2):
    B, S, Q, _ = adj_matrices.shape
    D = ids.shape[-1]
    T = 16
    NC = S // T

    b1r = b1.reshape(1, D)
    b2r = b2.reshape(1, D)

    body = functools.partial(_body, T=T, NC=NC)
    out = pl.pallas_call(
        body,
        grid=(NC,),
        in_specs=[
            pl.BlockSpec((B, T, Q, Q), lambda j: (0, NC - 1 - j, 0, 0)),
            pl.BlockSpec((Q, D), lambda j: (0, 0)),
            pl.BlockSpec((D, D), lambda j: (0, 0)),
            pl.BlockSpec((1, D), lambda j: (0, 0)),
            pl.BlockSpec((D, D), lambda j: (0, 0)),
            pl.BlockSpec((1, D), lambda j: (0, 0)),
        ],
        out_specs=pl.BlockSpec((B, T, Q, D), lambda j: (0, NC - 1 - j, 0, 0)),
        out_shape=jax.ShapeDtypeStruct((B, S, Q, D), jnp.float32),
        scratch_shapes=[
            pltpu.VMEM((B, Q, Q), jnp.float32),
            pltpu.VMEM((B, T, Q, Q), jnp.float32),
        ],
        compiler_params=pltpu.CompilerParams(
            dimension_semantics=("arbitrary",),
        ),
    )(adj_matrices, ids, W1, b1r, W2, b2r)
    return out
